# baseline (speedup 1.0000x reference)
import jax
import jax.numpy as jnp
from jax import lax
from jax.experimental import pallas as pl
from jax.experimental.pallas import tpu as pltpu


def _largest_divisor_leq(n, target):
    for b in range(min(target, n) // 8 * 8, 0, -8):
        if n % b == 0:
            return b
    return n


def kernel(input, adj, W):
    K, N, D_in = input.shape
    D_out = W.shape[1]

    bs = _largest_divisor_leq(N, 2000)
    S = N // bs
    bi = _largest_divisor_leq(N, 400)
    G = N // bi
    NBUF = 2

    def body(x_ref, adj_hbm, w_ref, out_ref, sup_ref, adj_bufs, adj_sems):
        i = pl.program_id(0)

        @pl.when(i == 0)
        def _prefetch():
            for c in range(NBUF):
                pltpu.make_async_copy(
                    adj_hbm.at[pl.ds(c * bi, bi)],
                    adj_bufs.at[c],
                    adj_sems.at[c],
                ).start()

        @pl.when(i < S)
        def _support():
            xx = x_ref[...].reshape(K * bs, D_in)
            acc = jnp.dot(xx, w_ref[...], preferred_element_type=jnp.float32)
            accb = acc.astype(jnp.bfloat16)
            for k in range(K):
                sup_ref[pl.ds(i * bs, bs), k * D_out:(k + 1) * D_out] = (
                    accb[k * bs:(k + 1) * bs])

        @pl.when(i >= S)
        def _spmm():
            g = i - S
            slot = lax.rem(g, NBUF)
            pltpu.make_async_copy(
                adj_hbm.at[pl.ds(g * bi, bi)],
                adj_bufs.at[slot],
                adj_sems.at[slot],
            ).wait()
            a = adj_bufs[slot].astype(jnp.bfloat16)
            acc = jnp.dot(a, sup_ref[...], preferred_element_type=jnp.float32)
            acc = jnp.maximum(acc, 0.0)
            for k in range(K):
                out_ref[k] = acc[:, k * D_out:(k + 1) * D_out]

            @pl.when(g + NBUF < G)
            def _refill():
                pltpu.make_async_copy(
                    adj_hbm.at[pl.ds((g + NBUF) * bi, bi)],
                    adj_bufs.at[slot],
                    adj_sems.at[slot],
                ).start()

    out = pl.pallas_call(
        body,
        grid=(S + G,),
        in_specs=[
            pl.BlockSpec((K, bs, D_in), lambda i: (0, jnp.minimum(i, S - 1), 0)),
            pl.BlockSpec(memory_space=pltpu.HBM),
            pl.BlockSpec((D_in, D_out), lambda i: (0, 0)),
        ],
        out_specs=pl.BlockSpec(
            (K, bi, D_out), lambda i: (0, jnp.maximum(i - S, 0), 0)),
        out_shape=jax.ShapeDtypeStruct((K, N, D_out), jnp.float32),
        scratch_shapes=[
            pltpu.VMEM((N, K * D_out), jnp.bfloat16),
            pltpu.VMEM((NBUF, bi, N), jnp.float32),
            pltpu.SemaphoreType.DMA((NBUF,)),
        ],
    )(input, adj, W)
    return out


# speedup vs baseline: 1.0339x; 1.0339x over previous
import jax
import jax.numpy as jnp
from jax import lax
from jax.experimental import pallas as pl
from jax.experimental.pallas import tpu as pltpu


def _largest_divisor_leq(n, target):
    for b in range(min(target, n) // 8 * 8, 0, -8):
        if n % b == 0:
            return b
    return n


def kernel(input, adj, W):
    K, N, D_in = input.shape
    D_out = W.shape[1]

    bs = _largest_divisor_leq(N, 2000)
    S = N // bs
    bi = _largest_divisor_leq(N, 400)
    G = N // bi
    NBUF = 2

    def body(x_ref, adj_hbm, w_ref, out_ref, sup_ref, adj_bufs, adj_sems):
        i = pl.program_id(0)

        @pl.when(i == 0)
        def _prefetch():
            pltpu.make_async_copy(
                adj_hbm.at[pl.ds(0, bi)],
                adj_bufs.at[0],
                adj_sems.at[0],
            ).start()

        for c in range(1, NBUF):
            @pl.when(i == S - NBUF + c)
            def _prefetch_late(c=c):
                pltpu.make_async_copy(
                    adj_hbm.at[pl.ds(c * bi, bi)],
                    adj_bufs.at[c],
                    adj_sems.at[c],
                ).start()

        @pl.when(i < S)
        def _support():
            xx = x_ref[...].reshape(K * bs, D_in)
            acc = jnp.dot(xx, w_ref[...], preferred_element_type=jnp.float32)
            accb = acc.astype(jnp.bfloat16)
            for k in range(K):
                sup_ref[pl.ds(i * bs, bs), k * D_out:(k + 1) * D_out] = (
                    accb[k * bs:(k + 1) * bs])

        @pl.when(i >= S)
        def _spmm():
            g = i - S
            slot = lax.rem(g, NBUF)
            pltpu.make_async_copy(
                adj_hbm.at[pl.ds(g * bi, bi)],
                adj_bufs.at[slot],
                adj_sems.at[slot],
            ).wait()
            a = adj_bufs[slot].astype(jnp.bfloat16)
            acc = jnp.dot(a, sup_ref[...], preferred_element_type=jnp.float32)
            acc = jnp.maximum(acc, 0.0)
            for k in range(K):
                out_ref[k] = acc[:, k * D_out:(k + 1) * D_out]

            @pl.when(g + NBUF < G)
            def _refill():
                pltpu.make_async_copy(
                    adj_hbm.at[pl.ds((g + NBUF) * bi, bi)],
                    adj_bufs.at[slot],
                    adj_sems.at[slot],
                ).start()

    out = pl.pallas_call(
        body,
        grid=(S + G,),
        in_specs=[
            pl.BlockSpec((K, bs, D_in), lambda i: (0, jnp.minimum(i, S - 1), 0)),
            pl.BlockSpec(memory_space=pltpu.HBM),
            pl.BlockSpec((D_in, D_out), lambda i: (0, 0)),
        ],
        out_specs=pl.BlockSpec(
            (K, bi, D_out), lambda i: (0, jnp.maximum(i - S, 0), 0)),
        out_shape=jax.ShapeDtypeStruct((K, N, D_out), jnp.float32),
        scratch_shapes=[
            pltpu.VMEM((N, K * D_out), jnp.bfloat16),
            pltpu.VMEM((NBUF, bi, N), jnp.float32),
            pltpu.SemaphoreType.DMA((NBUF,)),
        ],
    )(input, adj, W)
    return out
